# baseline (speedup 1.0000x reference)
import functools

import jax
import jax.numpy as jnp
from jax import lax
from jax.experimental import pallas as pl
from jax.experimental.pallas import tpu as pltpu
from jax.experimental.pallas import tpu_sc as plsc

N_COLS = 22
B = 4096
V = 100000
D = 64
NC, NS = 2, 16
NW = NC * NS
TOTAL = N_COLS * B
PER_W = TOTAL // NW
CHUNK = PER_W // 2
DG = 8
NG = D // DG
KG = N_COLS * DG
RG = 2 * KG
BT = 512
NBT = B // BT


def _sc_gather(table, flat_idx):
    mesh = plsc.VectorSubcoreMesh(core_axis_name="c", subcore_axis_name="s")

    @functools.partial(
        pl.kernel,
        out_type=jax.ShapeDtypeStruct((TOTAL, D), jnp.float32),
        mesh=mesh,
        compiler_params=pltpu.CompilerParams(use_tc_tiling_on_sc=False),
        scratch_types=[
            pltpu.VMEM((PER_W,), jnp.int32),
            pltpu.VMEM((CHUNK, D), jnp.float32),
            pltpu.SemaphoreType.DMA,
        ],
    )
    def gather_kernel(table_hbm, idx_hbm, out_hbm, idx_v, rows_v, sem):
        wid = lax.axis_index("s") * NC + lax.axis_index("c")
        base = wid * PER_W
        pltpu.sync_copy(idx_hbm.at[pl.ds(base, PER_W)], idx_v)
        for c in range(PER_W // CHUNK):
            pltpu.async_copy(
                table_hbm.at[idx_v.at[pl.ds(c * CHUNK, CHUNK)]], rows_v, sem
            ).wait()
            pltpu.sync_copy(rows_v, out_hbm.at[pl.ds(base + c * CHUNK, CHUNK)])

    return gather_kernel(table, flat_idx)


def _mlp_transpose_body(e_ref, w1_ref, b1_ref, w2_ref, b2_ref, out_ref):
    e = e_ref[0]
    acc = jnp.zeros_like(e) + b2_ref[0]
    for h in range(8):
        acc = acc + w2_ref[0, h] * jnp.tanh(e * w1_ref[h, 0] + b1_ref[h])
    t = acc.T
    out_ref[...] = t.reshape(NG, 1, DG, B)


def _pairwise_body(w_ref, gt_ref, lab_ref, pw_ref, inf_ref, loss_ref):
    bt = pl.program_id(0)
    g = pl.program_id(1)
    w = w_ref[g]
    gf = gt_ref[g]
    h = lax.dot_general(
        w, gf, (((1,), (0,)), ((), ())), preferred_element_type=jnp.float32,
        precision=lax.Precision.HIGHEST,
    )
    p0 = jnp.sum(h[0:KG, :] * gf, axis=0, keepdims=True)
    p1 = jnp.sum(h[KG:RG, :] * gf, axis=0, keepdims=True)
    contrib = jnp.concatenate([p0, p1], axis=0)

    @pl.when(g == 0)
    def _():
        inf_ref[...] = contrib

    @pl.when(g > 0)
    def _():
        inf_ref[...] = inf_ref[...] + contrib

    @pl.when(g == NG - 1)
    def _():
        inf = inf_ref[...]
        r = inf - lab_ref[...]
        partial = jnp.reshape(jnp.sum(pw_ref[...] * r * r) * (1.0 / B), (1, 1))
        prev = jnp.where(bt == 0, jnp.zeros((1, 1), jnp.float32), loss_ref[...])
        loss_ref[...] = prev + partial


def _build_wstack(W4):
    m4 = W4[:, 1, :, :].reshape(N_COLS, N_COLS, 2, D)
    w5 = m4.transpose(3, 2, 1, 0).reshape(NG, DG, 2, N_COLS, N_COLS)
    eye = jnp.eye(DG, dtype=w5.dtype)
    w6 = jnp.einsum("gdcji,de->gcjdie", w5, eye)
    return w6.reshape(NG, RG, KG)


def _tc_pipeline(e_flat, label, pos_weights, W4,
                 mlp_W1, mlp_b1, mlp_W2, mlp_b2):
    e3 = e_flat.reshape(N_COLS, B, D)
    smem = pl.BlockSpec(memory_space=pltpu.SMEM)
    gt4 = pl.pallas_call(
        _mlp_transpose_body,
        grid=(N_COLS,),
        in_specs=[
            pl.BlockSpec((1, B, D), lambda i: (i, 0, 0)),
            smem,
            smem,
            smem,
            smem,
        ],
        out_specs=pl.BlockSpec((NG, 1, DG, B), lambda i: (0, i, 0, 0)),
        out_shape=jax.ShapeDtypeStruct((NG, N_COLS, DG, B), jnp.float32),
    )(e3, mlp_W1, mlp_b1, mlp_W2, mlp_b2)
    gt = gt4.reshape(NG, KG, B)

    wstack = _build_wstack(W4)
    inf_t, loss = pl.pallas_call(
        _pairwise_body,
        grid=(NBT, NG),
        in_specs=[
            pl.BlockSpec((NG, RG, KG), lambda bt, g: (0, 0, 0)),
            pl.BlockSpec((NG, KG, BT), lambda bt, g: (0, 0, bt)),
            pl.BlockSpec((2, BT), lambda bt, g: (0, bt)),
            pl.BlockSpec((2, BT), lambda bt, g: (0, bt)),
        ],
        out_specs=[
            pl.BlockSpec((2, BT), lambda bt, g: (0, bt)),
            pl.BlockSpec((1, 1), lambda bt, g: (0, 0)),
        ],
        out_shape=[
            jax.ShapeDtypeStruct((2, B), jnp.float32),
            jax.ShapeDtypeStruct((1, 1), jnp.float32),
        ],
    )(wstack, gt, label.T, pos_weights.T)
    return inf_t.T, loss.reshape(())


def kernel(feature_indices, label, pos_weights, emb, W4, Wcat,
           mlp_W1, mlp_b1, mlp_W2, mlp_b2):
    offsets = (jnp.arange(N_COLS, dtype=jnp.int32) * V)[:, None]
    flat_idx = (feature_indices + offsets).reshape(TOTAL)
    table = emb.reshape(N_COLS * V, D)
    e_flat = _sc_gather(table, flat_idx)
    return _tc_pipeline(e_flat, label, pos_weights, W4,
                        mlp_W1, mlp_b1, mlp_W2, mlp_b2)


# speedup vs baseline: 1.4969x; 1.4969x over previous
import functools

import jax
import jax.numpy as jnp
from jax import lax
from jax.experimental import pallas as pl
from jax.experimental.pallas import tpu as pltpu
from jax.experimental.pallas import tpu_sc as plsc

N_COLS = 22
B = 4096
V = 100000
D = 64
NC, NS = 2, 16
NW = NC * NS
TOTAL = N_COLS * B
PER_W = TOTAL // NW
CHUNK = PER_W // 4
VT = 2048
HALF = 51200
DG = 8
NG = D // DG
KG = N_COLS * DG
RG = 2 * KG
BT = 512
NBT = B // BT


def _convert_body(x1_ref, x2_ref, o_ref):
    t1 = x1_ref[0].T
    t2 = x2_ref[0].T
    o_ref[0] = jnp.concatenate([t1, t2], axis=1)


def _convert_table(emb):
    et = jnp.swapaxes(emb, 1, 2)
    out = pl.pallas_call(
        _convert_body,
        grid=(N_COLS, HALF // VT),
        in_specs=[
            pl.BlockSpec((1, D, VT), lambda i, v: (i, 0, v)),
            pl.BlockSpec(
                (1, D, VT),
                lambda i, v: (i, 0, jnp.minimum(v + HALF // VT, (V - 1) // VT)),
            ),
        ],
        out_specs=pl.BlockSpec((1, VT, 2 * D), lambda i, v: (i, v, 0)),
        out_shape=jax.ShapeDtypeStruct((N_COLS, HALF, 2 * D), jnp.float32),
    )(et, et)
    return out.reshape(N_COLS * HALF, 2 * D)


def _sc_gather(table, pair_idx):
    mesh = plsc.VectorSubcoreMesh(core_axis_name="c", subcore_axis_name="s")

    @functools.partial(
        pl.kernel,
        out_type=jax.ShapeDtypeStruct((TOTAL, 2 * D), jnp.float32),
        mesh=mesh,
        scratch_types=[
            pltpu.VMEM((PER_W,), jnp.int32),
            pltpu.VMEM((CHUNK, 2 * D), jnp.float32),
            pltpu.SemaphoreType.DMA,
        ],
    )
    def gather_kernel(table_hbm, idx_hbm, out_hbm, idx_v, rows_v, sem):
        wid = lax.axis_index("s") * NC + lax.axis_index("c")
        base = wid * PER_W
        pltpu.sync_copy(idx_hbm.at[pl.ds(base, PER_W)], idx_v)
        for c in range(PER_W // CHUNK):
            pltpu.async_copy(
                table_hbm.at[idx_v.at[pl.ds(c * CHUNK, CHUNK)]], rows_v, sem
            ).wait()
            pltpu.sync_copy(rows_v, out_hbm.at[pl.ds(base + c * CHUNK, CHUNK)])

    return gather_kernel(table, pair_idx)


def _mlp_transpose_body(e_ref, par_ref, w1_ref, b1_ref, w2_ref, b2_ref, out_ref):
    e2 = e_ref[0]
    par = par_ref[0]
    e = jnp.where(par > 0.5, e2[:, D:2 * D], e2[:, 0:D])
    acc = jnp.zeros_like(e) + b2_ref[0]
    for h in range(8):
        acc = acc + w2_ref[0, h] * jnp.tanh(e * w1_ref[h, 0] + b1_ref[h])
    t = acc.T
    out_ref[...] = t.reshape(NG, 1, DG, B)


def _pairwise_body(w_ref, gt_ref, lab_ref, pw_ref, inf_ref, loss_ref):
    bt = pl.program_id(0)
    g = pl.program_id(1)
    w = w_ref[g]
    gf = gt_ref[g]
    h = lax.dot_general(
        w, gf, (((1,), (0,)), ((), ())), preferred_element_type=jnp.float32,
        precision=lax.Precision.HIGHEST,
    )
    p0 = jnp.sum(h[0:KG, :] * gf, axis=0, keepdims=True)
    p1 = jnp.sum(h[KG:RG, :] * gf, axis=0, keepdims=True)
    contrib = jnp.concatenate([p0, p1], axis=0)

    @pl.when(g == 0)
    def _():
        inf_ref[...] = contrib

    @pl.when(g > 0)
    def _():
        inf_ref[...] = inf_ref[...] + contrib

    @pl.when(g == NG - 1)
    def _():
        inf = inf_ref[...]
        r = inf - lab_ref[...]
        partial = jnp.reshape(jnp.sum(pw_ref[...] * r * r) * (1.0 / B), (1, 1))
        prev = jnp.where(bt == 0, jnp.zeros((1, 1), jnp.float32), loss_ref[...])
        loss_ref[...] = prev + partial


def _build_wstack(W4):
    m4 = W4[:, 1, :, :].reshape(N_COLS, N_COLS, 2, D)
    w5 = m4.transpose(3, 2, 1, 0).reshape(NG, DG, 2, N_COLS, N_COLS)
    eye = jnp.eye(DG, dtype=w5.dtype)
    w6 = jnp.einsum("gdcji,de->gcjdie", w5, eye)
    return w6.reshape(NG, RG, KG)


def _tc_pipeline(e_flat, parity, label, pos_weights, W4,
                 mlp_W1, mlp_b1, mlp_W2, mlp_b2):
    e3 = e_flat.reshape(N_COLS, B, 2 * D)
    smem = pl.BlockSpec(memory_space=pltpu.SMEM)
    gt4 = pl.pallas_call(
        _mlp_transpose_body,
        grid=(N_COLS,),
        in_specs=[
            pl.BlockSpec((1, B, 2 * D), lambda i: (i, 0, 0)),
            pl.BlockSpec((1, B, 1), lambda i: (i, 0, 0)),
            smem,
            smem,
            smem,
            smem,
        ],
        out_specs=pl.BlockSpec((NG, 1, DG, B), lambda i: (0, i, 0, 0)),
        out_shape=jax.ShapeDtypeStruct((NG, N_COLS, DG, B), jnp.float32),
    )(e3, parity, mlp_W1, mlp_b1, mlp_W2, mlp_b2)
    gt = gt4.reshape(NG, KG, B)

    wstack = _build_wstack(W4)
    inf_t, loss = pl.pallas_call(
        _pairwise_body,
        grid=(NBT, NG),
        in_specs=[
            pl.BlockSpec((NG, RG, KG), lambda bt, g: (0, 0, 0)),
            pl.BlockSpec((NG, KG, BT), lambda bt, g: (0, 0, bt)),
            pl.BlockSpec((2, BT), lambda bt, g: (0, bt)),
            pl.BlockSpec((2, BT), lambda bt, g: (0, bt)),
        ],
        out_specs=[
            pl.BlockSpec((2, BT), lambda bt, g: (0, bt)),
            pl.BlockSpec((1, 1), lambda bt, g: (0, 0)),
        ],
        out_shape=[
            jax.ShapeDtypeStruct((2, B), jnp.float32),
            jax.ShapeDtypeStruct((1, 1), jnp.float32),
        ],
    )(wstack, gt, label.T, pos_weights.T)
    return inf_t.T, loss.reshape(())


def kernel(feature_indices, label, pos_weights, emb, W4, Wcat,
           mlp_W1, mlp_b1, mlp_W2, mlp_b2):
    high = (feature_indices >= HALF).astype(jnp.int32)
    local_v = feature_indices - HALF * high
    offsets = (jnp.arange(N_COLS, dtype=jnp.int32) * HALF)[:, None]
    pair_idx = (local_v + offsets).reshape(TOTAL)
    parity = high.astype(jnp.float32).reshape(N_COLS, B, 1)
    table = _convert_table(emb)
    e_flat = _sc_gather(table, pair_idx)
    return _tc_pipeline(e_flat, parity, label, pos_weights, W4,
                        mlp_W1, mlp_b1, mlp_W2, mlp_b2)
